# baseline (speedup 1.0000x reference)
import functools

import jax
import jax.numpy as jnp
from jax import lax
from jax.experimental import pallas as pl
from jax.experimental.pallas import tpu as pltpu
from jax.experimental.pallas import tpu_sc as plsc

N = 10000
E = 320000
F_IN = 128
H1P = 16
H2P = 32
FC1 = 32
C = 10

N_PAD = 10240
N_SLICE = N_PAD // 16
CHUNK = 128
ECH = E // CHUNK
NCH_LO = ECH // 32
NCH_EXTRA = ECH - 32 * NCH_LO
NCH_HI = NCH_LO + 1

_MESH = plsc.VectorSubcoreMesh(core_axis_name="c", subcore_axis_name="s")
_SC_PARAMS = pltpu.CompilerParams(use_tc_tiling_on_sc=False)


_NBUF = 8
_NRING = 8
_LAG = 4


def _tile_range(tile):
    n = NCH_LO + jnp.where(tile < NCH_EXTRA, 1, 0)
    base = tile * NCH_LO + jnp.minimum(tile, NCH_EXTRA)
    return base, n


def _make_deg_kernel():
    @functools.partial(
        pl.kernel,
        out_type=jax.ShapeDtypeStruct((2, N_PAD), jnp.float32),
        mesh=_MESH,
        compiler_params=_SC_PARAMS,
        scratch_types=[
            pltpu.VMEM((NCH_HI, CHUNK), jnp.int32),
            pltpu.VMEM((CHUNK,), jnp.float32),
            pltpu.VMEM_SHARED((N_PAD,), jnp.float32),
            pltpu.SemaphoreType.DMA((_NBUF,)),
        ],
    )
    def deg_kernel(ei_hbm, ones_hbm, zeros_hbm, out_hbm, idx_d, ones_v, acc,
                   ssem):
        c = lax.axis_index("c")
        s = lax.axis_index("s")
        tile = c * 16 + s
        base, n_my = _tile_range(tile)
        pltpu.sync_copy(zeros_hbm.at[pl.ds(s * N_SLICE, N_SLICE)],
                        acc.at[pl.ds(s * N_SLICE, N_SLICE)])
        pltpu.sync_copy(ones_hbm, ones_v)
        pltpu.sync_copy(ei_hbm.at[1, pl.ds(base, NCH_LO)],
                        idx_d.at[pl.ds(0, NCH_LO)])

        @pl.when(tile < NCH_EXTRA)
        def _():
            pltpu.sync_copy(ei_hbm.at[1, base + NCH_LO], idx_d.at[NCH_LO])

        plsc.subcore_barrier()

        for b in range(_NBUF):
            pltpu.async_copy(ones_v, acc.at[idx_d.at[b]], ssem.at[b], add=True)

        @pl.loop(_NBUF, _NBUF * ((NCH_HI + _NBUF - 1) // _NBUF + 1), step=_NBUF)
        def _(g):
            for b in range(_NBUF):
                j = g + b

                @pl.when(j < n_my)
                def _():
                    pltpu.make_async_copy(
                        ones_v, acc.at[idx_d.at[0]], ssem.at[b]).wait()
                    pltpu.async_copy(ones_v, acc.at[idx_d.at[j]], ssem.at[b],
                                     add=True)

        for b in range(_NBUF):
            pltpu.make_async_copy(ones_v, acc.at[idx_d.at[0]], ssem.at[b]).wait()
        plsc.subcore_barrier()
        pltpu.sync_copy(acc.at[pl.ds(s * N_SLICE, N_SLICE)],
                        out_hbm.at[c, pl.ds(s * N_SLICE, N_SLICE)])

    return deg_kernel


def _make_agg_kernel(H):
    @functools.partial(
        pl.kernel,
        out_type=jax.ShapeDtypeStruct((2, N_PAD, H), jnp.float32),
        mesh=_MESH,
        compiler_params=_SC_PARAMS,
        scratch_types=[
            pltpu.VMEM((2, NCH_HI, CHUNK), jnp.int32),
            pltpu.VMEM((_NRING, CHUNK, H), jnp.float32),
            pltpu.VMEM_SHARED((N_PAD, H), jnp.float32),
            pltpu.SemaphoreType.DMA((_NRING,)),
            pltpu.SemaphoreType.DMA((_NRING,)),
        ],
    )
    def agg_kernel(ei_hbm, hs_hbm, zeros_hbm, out_hbm, idx, rows, acc,
                   gsem, ssem):
        c = lax.axis_index("c")
        s = lax.axis_index("s")
        tile = c * 16 + s
        base, n_my = _tile_range(tile)
        pltpu.sync_copy(zeros_hbm.at[pl.ds(s * N_SLICE, N_SLICE)],
                        acc.at[pl.ds(s * N_SLICE, N_SLICE)])
        for h in range(2):
            pltpu.sync_copy(ei_hbm.at[h, pl.ds(base, NCH_LO)],
                            idx.at[h, pl.ds(0, NCH_LO)])

        @pl.when(tile < NCH_EXTRA)
        def _():
            for h in range(2):
                pltpu.sync_copy(ei_hbm.at[h, base + NCH_LO], idx.at[h, NCH_LO])

        plsc.subcore_barrier()

        n_visits = _NRING * ((NCH_HI + 2 * _NRING - 1) // _NRING)

        @pl.loop(0, n_visits, step=_NRING)
        def _(g):
            for k in range(_NRING):
                v = g + k
                b_s = (k + _NRING - _LAG) % _NRING

                @pl.when(jnp.logical_and(v >= _NRING, v - _NRING < n_my))
                def _():
                    pltpu.make_async_copy(
                        rows.at[k], acc.at[idx.at[1, 0]], ssem.at[k]).wait()

                @pl.when(v < n_my)
                def _():
                    pltpu.async_copy(hs_hbm.at[idx.at[0, v]], rows.at[k],
                                     gsem.at[k])

                @pl.when(jnp.logical_and(v >= _LAG, v - _LAG < n_my))
                def _():
                    pltpu.make_async_copy(
                        hs_hbm.at[idx.at[0, 0]], rows.at[b_s],
                        gsem.at[b_s]).wait()
                    pltpu.async_copy(rows.at[b_s], acc.at[idx.at[1, v - _LAG]],
                                     ssem.at[b_s], add=True)

        plsc.subcore_barrier()
        pltpu.sync_copy(acc.at[pl.ds(s * N_SLICE, N_SLICE)],
                        out_hbm.at[c, pl.ds(s * N_SLICE, N_SLICE)])

    return agg_kernel


_deg_kernel = _make_deg_kernel()
_agg16 = _make_agg_kernel(H1P)
_agg32 = _make_agg_kernel(H2P)


_RB = 2048


def _tc1_body(x_ref, w_ref, deg_ref, hs_ref, dinv_ref):
    deg = deg_ref[0, :] + deg_ref[1, :] + 1.0
    dinv = lax.rsqrt(deg)
    h = jnp.dot(x_ref[...], w_ref[...], preferred_element_type=jnp.float32)
    hs_ref[...] = h * dinv[:, None]
    dinv_ref[...] = dinv


def _tc1(x, W1p, deg2):
    grid = (N_PAD // _RB,)
    return pl.pallas_call(
        _tc1_body,
        grid=grid,
        in_specs=[
            pl.BlockSpec((_RB, F_IN), lambda i: (i, 0)),
            pl.BlockSpec((F_IN, H1P), lambda i: (0, 0)),
            pl.BlockSpec((2, _RB), lambda i: (0, i)),
        ],
        out_specs=[
            pl.BlockSpec((_RB, H1P), lambda i: (i, 0)),
            pl.BlockSpec((_RB,), lambda i: (i,)),
        ],
        out_shape=[
            jax.ShapeDtypeStruct((N_PAD, H1P), jnp.float32),
            jax.ShapeDtypeStruct((N_PAD,), jnp.float32),
        ],
    )(x, W1p, deg2)


def _tc2_body(p_ref, hs_ref, dinv_ref, b_ref, w_ref, out_ref):
    dinv = dinv_ref[...]
    g = (p_ref[0] + p_ref[1] + hs_ref[...]) * dinv[:, None] + b_ref[...][None, :]
    g = jnp.maximum(g, 0.0)
    h2 = jnp.dot(g, w_ref[...], preferred_element_type=jnp.float32)
    out_ref[...] = h2 * dinv[:, None]


def _tc2(parts1, hs1, dinv, b1p, W2p):
    grid = (N_PAD // _RB,)
    return pl.pallas_call(
        _tc2_body,
        grid=grid,
        in_specs=[
            pl.BlockSpec((2, _RB, H1P), lambda i: (0, i, 0)),
            pl.BlockSpec((_RB, H1P), lambda i: (i, 0)),
            pl.BlockSpec((_RB,), lambda i: (i,)),
            pl.BlockSpec((H1P,), lambda i: (0,)),
            pl.BlockSpec((H1P, H2P), lambda i: (0, 0)),
        ],
        out_specs=pl.BlockSpec((_RB, H2P), lambda i: (i, 0)),
        out_shape=jax.ShapeDtypeStruct((N_PAD, H2P), jnp.float32),
    )(parts1, hs1, dinv, b1p, W2p)


def _tc3_body(p_ref, hs_ref, dinv_ref, b2_ref, wf1_ref, bf1_ref, wf2_ref,
              bf2_ref, out_ref):
    dinv = dinv_ref[...]
    g = (p_ref[0] + p_ref[1] + hs_ref[...]) * dinv[:, None] + b2_ref[...][None, :]
    g = jnp.maximum(g, 0.0)
    t = jnp.dot(g, wf1_ref[...], preferred_element_type=jnp.float32)
    t = jnp.maximum(t + bf1_ref[...][None, :], 0.0)
    z = jnp.dot(t, wf2_ref[...], preferred_element_type=jnp.float32)
    out_ref[...] = z + bf2_ref[...][None, :]


def _tc3(parts2, hs2, dinv, b2p, Wf1p, bf1, Wf2p, bf2p):
    grid = (N_PAD // _RB,)
    return pl.pallas_call(
        _tc3_body,
        grid=grid,
        in_specs=[
            pl.BlockSpec((2, _RB, H2P), lambda i: (0, i, 0)),
            pl.BlockSpec((_RB, H2P), lambda i: (i, 0)),
            pl.BlockSpec((_RB,), lambda i: (i,)),
            pl.BlockSpec((H2P,), lambda i: (0,)),
            pl.BlockSpec((H2P, FC1), lambda i: (0, 0)),
            pl.BlockSpec((FC1,), lambda i: (0,)),
            pl.BlockSpec((FC1, 16), lambda i: (0, 0)),
            pl.BlockSpec((16,), lambda i: (0,)),
        ],
        out_specs=pl.BlockSpec((_RB, 16), lambda i: (i, 0)),
        out_shape=jax.ShapeDtypeStruct((N_PAD, 16), jnp.float32),
    )(parts2, hs2, dinv, b2p, Wf1p, bf1, Wf2p, bf2p)


def kernel(x, edge_index, W1, b1, W2, b2, Wf1, bf1, Wf2, bf2):
    ei = edge_index.reshape(2, ECH, CHUNK)

    xp = jnp.zeros((N_PAD, F_IN), jnp.float32).at[:N].set(x)
    W1p = jnp.zeros((F_IN, H1P), jnp.float32).at[:, :W1.shape[1]].set(W1)
    b1p = jnp.zeros((H1P,), jnp.float32).at[:b1.shape[0]].set(b1)
    W2p = jnp.zeros((H1P, H2P), jnp.float32).at[:W2.shape[0], :W2.shape[1]].set(W2)
    b2p = jnp.zeros((H2P,), jnp.float32).at[:b2.shape[0]].set(b2)
    Wf1p = jnp.zeros((H2P, FC1), jnp.float32).at[:Wf1.shape[0]].set(Wf1)
    Wf2p = jnp.zeros((FC1, 16), jnp.float32).at[:, :Wf2.shape[1]].set(Wf2)
    bf2p = jnp.zeros((16,), jnp.float32).at[:bf2.shape[0]].set(bf2)

    ones128 = jnp.ones((CHUNK,), jnp.float32)
    zeros1 = jnp.zeros((N_PAD,), jnp.float32)
    zeros16 = jnp.zeros((N_PAD, H1P), jnp.float32)
    zeros32 = jnp.zeros((N_PAD, H2P), jnp.float32)

    deg2 = _deg_kernel(ei, ones128, zeros1)
    hs1, dinv = _tc1(xp, W1p, deg2)
    parts1 = _agg16(ei, hs1, zeros16)
    hs2 = _tc2(parts1, hs1, dinv, b1p, W2p)
    parts2 = _agg32(ei, hs2, zeros32)
    z = _tc3(parts2, hs2, dinv, b2p, Wf1p, bf1, Wf2p, bf2p)
    return z[:N, :C]


# speedup vs baseline: 1.0364x; 1.0364x over previous
import functools

import jax
import jax.numpy as jnp
from jax import lax
from jax.experimental import pallas as pl
from jax.experimental.pallas import tpu as pltpu
from jax.experimental.pallas import tpu_sc as plsc

N = 10000
E = 320000
F_IN = 128
H1P = 16
H2P = 32
FC1 = 32
C = 10

N_PAD = 10240
N_SLICE = N_PAD // 16
CHUNK = 128
ECH = E // CHUNK
NCH_LO = ECH // 32
NCH_EXTRA = ECH - 32 * NCH_LO
NCH_HI = NCH_LO + 1

_MESH = plsc.VectorSubcoreMesh(core_axis_name="c", subcore_axis_name="s")
_SC_PARAMS = pltpu.CompilerParams(use_tc_tiling_on_sc=False)


_NBUF = 8


def _tile_range(tile):
    n = NCH_LO + jnp.where(tile < NCH_EXTRA, 1, 0)
    base = tile * NCH_LO + jnp.minimum(tile, NCH_EXTRA)
    return base, n


def _make_deg_kernel():
    @functools.partial(
        pl.kernel,
        out_type=jax.ShapeDtypeStruct((2, N_PAD), jnp.float32),
        mesh=_MESH,
        compiler_params=_SC_PARAMS,
        scratch_types=[
            pltpu.VMEM((NCH_HI, CHUNK), jnp.int32),
            pltpu.VMEM((CHUNK,), jnp.float32),
            pltpu.VMEM_SHARED((N_PAD,), jnp.float32),
            pltpu.SemaphoreType.DMA((_NBUF,)),
        ],
    )
    def deg_kernel(ei_hbm, ones_hbm, zeros_hbm, out_hbm, idx_d, ones_v, acc,
                   ssem):
        c = lax.axis_index("c")
        s = lax.axis_index("s")
        tile = c * 16 + s
        base, n_my = _tile_range(tile)
        pltpu.sync_copy(zeros_hbm.at[pl.ds(s * N_SLICE, N_SLICE)],
                        acc.at[pl.ds(s * N_SLICE, N_SLICE)])
        pltpu.sync_copy(ones_hbm, ones_v)
        pltpu.sync_copy(ei_hbm.at[1, pl.ds(base, NCH_LO)],
                        idx_d.at[pl.ds(0, NCH_LO)])

        @pl.when(tile < NCH_EXTRA)
        def _():
            pltpu.sync_copy(ei_hbm.at[1, base + NCH_LO], idx_d.at[NCH_LO])

        plsc.subcore_barrier()

        for b in range(_NBUF):
            pltpu.async_copy(ones_v, acc.at[idx_d.at[b]], ssem.at[b], add=True)

        @pl.loop(_NBUF, _NBUF * ((NCH_HI + _NBUF - 1) // _NBUF + 1), step=_NBUF)
        def _(g):
            for b in range(_NBUF):
                j = g + b

                @pl.when(j < n_my)
                def _():
                    pltpu.make_async_copy(
                        ones_v, acc.at[idx_d.at[0]], ssem.at[b]).wait()
                    pltpu.async_copy(ones_v, acc.at[idx_d.at[j]], ssem.at[b],
                                     add=True)

        for b in range(_NBUF):
            pltpu.make_async_copy(ones_v, acc.at[idx_d.at[0]], ssem.at[b]).wait()
        plsc.subcore_barrier()
        pltpu.sync_copy(acc.at[pl.ds(s * N_SLICE, N_SLICE)],
                        out_hbm.at[c, pl.ds(s * N_SLICE, N_SLICE)])

    return deg_kernel


def _make_agg_kernel(H):
    @functools.partial(
        pl.kernel,
        out_type=jax.ShapeDtypeStruct((2, N_PAD, H), jnp.float32),
        mesh=_MESH,
        compiler_params=_SC_PARAMS,
        scratch_types=[
            pltpu.VMEM((2, NCH_HI, CHUNK), jnp.int32),
            pltpu.VMEM((_NBUF, CHUNK, H), jnp.float32),
            pltpu.VMEM_SHARED((N_PAD, H), jnp.float32),
            pltpu.SemaphoreType.DMA((_NBUF,)),
        ],
    )
    def agg_kernel(ei_hbm, hs_hbm, zeros_hbm, out_hbm, idx, rows, acc, gsem):
        c = lax.axis_index("c")
        s = lax.axis_index("s")
        tile = c * 16 + s
        base, n_my = _tile_range(tile)
        pltpu.sync_copy(zeros_hbm.at[pl.ds(s * N_SLICE, N_SLICE)],
                        acc.at[pl.ds(s * N_SLICE, N_SLICE)])
        for h in range(2):
            pltpu.sync_copy(ei_hbm.at[h, pl.ds(base, NCH_LO)],
                            idx.at[h, pl.ds(0, NCH_LO)])

        @pl.when(tile < NCH_EXTRA)
        def _():
            for h in range(2):
                pltpu.sync_copy(ei_hbm.at[h, base + NCH_LO], idx.at[h, NCH_LO])

        plsc.subcore_barrier()

        for b in range(_NBUF):
            pltpu.async_copy(hs_hbm.at[idx.at[0, b]], rows.at[b], gsem.at[b])

        @pl.loop(0, _NBUF * ((NCH_HI + _NBUF - 1) // _NBUF), step=_NBUF)
        def _(g):
            for b in range(_NBUF):
                j = g + b

                @pl.when(j < n_my)
                def _():
                    pltpu.make_async_copy(
                        hs_hbm.at[idx.at[0, 0]], rows.at[b], gsem.at[b]).wait()
                    pltpu.sync_copy(rows.at[b], acc.at[idx.at[1, j]], add=True)

                    @pl.when(j + _NBUF < n_my)
                    def _():
                        pltpu.async_copy(hs_hbm.at[idx.at[0, j + _NBUF]],
                                         rows.at[b], gsem.at[b])

        plsc.subcore_barrier()
        pltpu.sync_copy(acc.at[pl.ds(s * N_SLICE, N_SLICE)],
                        out_hbm.at[c, pl.ds(s * N_SLICE, N_SLICE)])

    return agg_kernel


_deg_kernel = _make_deg_kernel()
_agg16 = _make_agg_kernel(H1P)
_agg32 = _make_agg_kernel(H2P)


_RB = 2048


def _tc1_body(x_ref, w_ref, deg_ref, hs_ref, dinv_ref):
    deg = deg_ref[0, :] + deg_ref[1, :] + 1.0
    dinv = lax.rsqrt(deg)
    h = jnp.dot(x_ref[...], w_ref[...], preferred_element_type=jnp.float32)
    hs_ref[...] = h * dinv[:, None]
    dinv_ref[...] = dinv


def _tc1(x, W1p, deg2):
    grid = (N_PAD // _RB,)
    return pl.pallas_call(
        _tc1_body,
        grid=grid,
        in_specs=[
            pl.BlockSpec((_RB, F_IN), lambda i: (i, 0)),
            pl.BlockSpec((F_IN, H1P), lambda i: (0, 0)),
            pl.BlockSpec((2, _RB), lambda i: (0, i)),
        ],
        out_specs=[
            pl.BlockSpec((_RB, H1P), lambda i: (i, 0)),
            pl.BlockSpec((_RB,), lambda i: (i,)),
        ],
        out_shape=[
            jax.ShapeDtypeStruct((N_PAD, H1P), jnp.float32),
            jax.ShapeDtypeStruct((N_PAD,), jnp.float32),
        ],
    )(x, W1p, deg2)


def _tc2_body(p_ref, hs_ref, dinv_ref, b_ref, w_ref, out_ref):
    dinv = dinv_ref[...]
    g = (p_ref[0] + p_ref[1] + hs_ref[...]) * dinv[:, None] + b_ref[...][None, :]
    g = jnp.maximum(g, 0.0)
    h2 = jnp.dot(g, w_ref[...], preferred_element_type=jnp.float32)
    out_ref[...] = h2 * dinv[:, None]


def _tc2(parts1, hs1, dinv, b1p, W2p):
    grid = (N_PAD // _RB,)
    return pl.pallas_call(
        _tc2_body,
        grid=grid,
        in_specs=[
            pl.BlockSpec((2, _RB, H1P), lambda i: (0, i, 0)),
            pl.BlockSpec((_RB, H1P), lambda i: (i, 0)),
            pl.BlockSpec((_RB,), lambda i: (i,)),
            pl.BlockSpec((H1P,), lambda i: (0,)),
            pl.BlockSpec((H1P, H2P), lambda i: (0, 0)),
        ],
        out_specs=pl.BlockSpec((_RB, H2P), lambda i: (i, 0)),
        out_shape=jax.ShapeDtypeStruct((N_PAD, H2P), jnp.float32),
    )(parts1, hs1, dinv, b1p, W2p)


def _tc3_body(p_ref, hs_ref, dinv_ref, b2_ref, wf1_ref, bf1_ref, wf2_ref,
              bf2_ref, out_ref):
    dinv = dinv_ref[...]
    g = (p_ref[0] + p_ref[1] + hs_ref[...]) * dinv[:, None] + b2_ref[...][None, :]
    g = jnp.maximum(g, 0.0)
    t = jnp.dot(g, wf1_ref[...], preferred_element_type=jnp.float32)
    t = jnp.maximum(t + bf1_ref[...][None, :], 0.0)
    z = jnp.dot(t, wf2_ref[...], preferred_element_type=jnp.float32)
    out_ref[...] = z + bf2_ref[...][None, :]


def _tc3(parts2, hs2, dinv, b2p, Wf1p, bf1, Wf2p, bf2p):
    grid = (N_PAD // _RB,)
    return pl.pallas_call(
        _tc3_body,
        grid=grid,
        in_specs=[
            pl.BlockSpec((2, _RB, H2P), lambda i: (0, i, 0)),
            pl.BlockSpec((_RB, H2P), lambda i: (i, 0)),
            pl.BlockSpec((_RB,), lambda i: (i,)),
            pl.BlockSpec((H2P,), lambda i: (0,)),
            pl.BlockSpec((H2P, FC1), lambda i: (0, 0)),
            pl.BlockSpec((FC1,), lambda i: (0,)),
            pl.BlockSpec((FC1, 16), lambda i: (0, 0)),
            pl.BlockSpec((16,), lambda i: (0,)),
        ],
        out_specs=pl.BlockSpec((_RB, 16), lambda i: (i, 0)),
        out_shape=jax.ShapeDtypeStruct((N_PAD, 16), jnp.float32),
    )(parts2, hs2, dinv, b2p, Wf1p, bf1, Wf2p, bf2p)


def kernel(x, edge_index, W1, b1, W2, b2, Wf1, bf1, Wf2, bf2):
    ei = edge_index.reshape(2, ECH, CHUNK)

    xp = jnp.zeros((N_PAD, F_IN), jnp.float32).at[:N].set(x)
    W1p = jnp.zeros((F_IN, H1P), jnp.float32).at[:, :W1.shape[1]].set(W1)
    b1p = jnp.zeros((H1P,), jnp.float32).at[:b1.shape[0]].set(b1)
    W2p = jnp.zeros((H1P, H2P), jnp.float32).at[:W2.shape[0], :W2.shape[1]].set(W2)
    b2p = jnp.zeros((H2P,), jnp.float32).at[:b2.shape[0]].set(b2)
    Wf1p = jnp.zeros((H2P, FC1), jnp.float32).at[:Wf1.shape[0]].set(Wf1)
    Wf2p = jnp.zeros((FC1, 16), jnp.float32).at[:, :Wf2.shape[1]].set(Wf2)
    bf2p = jnp.zeros((16,), jnp.float32).at[:bf2.shape[0]].set(bf2)

    ones128 = jnp.ones((CHUNK,), jnp.float32)
    zeros1 = jnp.zeros((N_PAD,), jnp.float32)
    zeros16 = jnp.zeros((N_PAD, H1P), jnp.float32)
    zeros32 = jnp.zeros((N_PAD, H2P), jnp.float32)

    deg2 = _deg_kernel(ei, ones128, zeros1)
    hs1, dinv = _tc1(xp, W1p, deg2)
    parts1 = _agg16(ei, hs1, zeros16)
    hs2 = _tc2(parts1, hs1, dinv, b1p, W2p)
    parts2 = _agg32(ei, hs2, zeros32)
    z = _tc3(parts2, hs2, dinv, b2p, Wf1p, bf1, Wf2p, bf2p)
    return z[:N, :C]
